# Initial kernel scaffold; baseline (speedup 1.0000x reference)
import functools

import jax
import jax.numpy as jnp
from jax import lax
from jax.experimental import pallas as pl
from jax.experimental.pallas import tpu as pltpu
from jax.experimental.pallas import tpu_sc as plsc

D = 32
B_TOTAL = 16384 * 50
NC, NS = 2, 16
NW = NC * NS
B_PER_W = B_TOTAL // NW
CHUNK = 1600
NCHUNK = B_PER_W // CHUNK


def _make_kernel():
    mesh = plsc.VectorSubcoreMesh(core_axis_name="c", subcore_axis_name="s")

    @functools.partial(
        pl.kernel,
        mesh=mesh,
        out_type=jax.ShapeDtypeStruct((B_TOTAL, D), jnp.float32),
        scratch_types=[
            pltpu.VMEM((1, CHUNK), jnp.int32),
            pltpu.VMEM((1, CHUNK, D), jnp.float32),
            pltpu.SemaphoreType.DMA,
        ],
    )
    def gather_kernel(idx_hbm, table_hbm, out_hbm, idx_v, rows_v, sem):
        wid = lax.axis_index("s") * NC + lax.axis_index("c")
        base = wid * B_PER_W

        def body(g, carry):
            off = base + g * CHUNK
            pltpu.sync_copy(idx_hbm.at[pl.ds(off, CHUNK)], idx_v.at[0])
            pltpu.async_copy(table_hbm.at[idx_v.at[0]], rows_v.at[0], sem).wait()
            pltpu.sync_copy(rows_v.at[0], out_hbm.at[pl.ds(off, CHUNK)])
            return carry

        lax.fori_loop(0, NCHUNK, body, 0)

    return gather_kernel


_GATHER = _make_kernel()


def kernel(element_ids, weight):
    idx = element_ids.reshape(-1)
    out = _GATHER(idx, weight)
    return out.reshape(element_ids.shape + (D,))


# speedup vs baseline: 1.1099x; 1.1099x over previous
import functools

import jax
import jax.numpy as jnp
from jax import lax
from jax.experimental import pallas as pl
from jax.experimental.pallas import tpu as pltpu
from jax.experimental.pallas import tpu_sc as plsc

D = 32
B_TOTAL = 16384 * 50
NC, NS = 2, 16
NW = NC * NS
B_PER_W = B_TOTAL // NW
CHUNK = 1600
NCHUNK = B_PER_W // CHUNK
NBUF = 2


def _make_kernel():
    mesh = plsc.VectorSubcoreMesh(core_axis_name="c", subcore_axis_name="s")

    @functools.partial(
        pl.kernel,
        mesh=mesh,
        out_type=jax.ShapeDtypeStruct((B_TOTAL, D), jnp.float32),
        scratch_types=[
            pltpu.VMEM((NBUF, CHUNK), jnp.int32),
            pltpu.VMEM((NBUF, CHUNK, D), jnp.float32),
            [pltpu.SemaphoreType.DMA] * NBUF,
            [pltpu.SemaphoreType.DMA] * NBUF,
        ],
        compiler_params=pltpu.CompilerParams(use_tc_tiling_on_sc=False),
    )
    def gather_kernel(idx_hbm, table_hbm, out_hbm, idx_v, rows_v, gsem, wsem):
        wid = lax.axis_index("s") * NC + lax.axis_index("c")
        base = wid * B_PER_W

        def start_gather(g):
            b = g % NBUF
            off = base + g * CHUNK
            pltpu.sync_copy(idx_hbm.at[pl.ds(off, CHUNK)], idx_v.at[b])
            return pltpu.async_copy(table_hbm.at[idx_v.at[b]], rows_v.at[b],
                                    gsem[b])

        gathers = {0: start_gather(0)}
        writes = {}
        for g in range(NCHUNK):
            b = g % NBUF
            if g + 1 < NCHUNK:
                if g - 1 >= 0:
                    writes[g - 1].wait()
                gathers[g + 1] = start_gather(g + 1)
            gathers[g].wait()
            off = base + g * CHUNK
            writes[g] = pltpu.async_copy(rows_v.at[b],
                                         out_hbm.at[pl.ds(off, CHUNK)],
                                         wsem[b])
        writes[NCHUNK - 2].wait()
        writes[NCHUNK - 1].wait()

    return gather_kernel


_GATHER = _make_kernel()


def kernel(element_ids, weight):
    idx = element_ids.reshape(-1)
    out = _GATHER(idx, weight)
    return out.reshape(element_ids.shape + (D,))


# speedup vs baseline: 1.3774x; 1.2410x over previous
import functools

import jax
import jax.numpy as jnp
from jax import lax
from jax.experimental import pallas as pl
from jax.experimental.pallas import tpu as pltpu
from jax.experimental.pallas import tpu_sc as plsc

D = 32
B = 16384
S = 50
NC, NS = 2, 16
NW = NC * NS
BW = B // NW
FW = BW * S


def _make_kernel():
    mesh = plsc.VectorSubcoreMesh(core_axis_name="c", subcore_axis_name="s")

    @functools.partial(
        pl.kernel,
        mesh=mesh,
        out_type=jax.ShapeDtypeStruct((S, D, B), jnp.float32),
        scratch_types=[
            pltpu.VMEM((FW,), jnp.int32),
            pltpu.VMEM((BW,), jnp.int32),
            pltpu.VMEM((BW, D), jnp.float32),
            pltpu.VMEM((D, BW), jnp.float32),
            pltpu.SemaphoreType.DMA,
        ],
        compiler_params=pltpu.CompilerParams(
            use_tc_tiling_on_sc=False, needs_layout_passes=False
        ),
    )
    def gather_kernel(idx_hbm, table_hbm, out_hbm, ids_v, sidx_v, d_v, e_v, sem):
        wid = lax.axis_index("s") * NC + lax.axis_index("c")
        b0 = wid * BW
        pltpu.sync_copy(idx_hbm.at[pl.ds(wid * FW, FW)], ids_v)

        iota = lax.iota(jnp.int32, 16)
        rvecs = [iota + 16 * g for g in range(BW // 16)]
        dvecs = [jnp.full((16,), d, jnp.int32) for d in range(D)]

        def body(s, carry):
            for g in range(BW // 16):
                vis = plsc.load_gather(ids_v, [rvecs[g] * S + s])
                sidx_v[pl.ds(16 * g, 16)] = vis
            pltpu.async_copy(table_hbm.at[sidx_v], d_v, sem).wait()
            for g in range(BW // 16):
                for d in range(D):
                    x = plsc.load_gather(d_v, [rvecs[g], dvecs[d]])
                    e_v[d, pl.ds(16 * g, 16)] = x
            pltpu.sync_copy(e_v, out_hbm.at[s].at[:, pl.ds(b0, BW)])
            return carry

        lax.fori_loop(0, S, body, 0)

    return gather_kernel


_GATHER = _make_kernel()


def kernel(element_ids, weight):
    idx = element_ids.reshape(-1)
    out = _GATHER(idx, weight)
    return out.transpose(2, 0, 1)


# speedup vs baseline: 1.9915x; 1.4459x over previous
import functools

import jax
import jax.numpy as jnp
from jax import lax
from jax.experimental import pallas as pl
from jax.experimental.pallas import tpu as pltpu
from jax.experimental.pallas import tpu_sc as plsc

D = 32
B = 16384
S = 50
NC, NS = 2, 16
NW = NC * NS
BW = B // NW
FW = BW * S


def _make_kernel():
    mesh = plsc.VectorSubcoreMesh(core_axis_name="c", subcore_axis_name="s")

    @functools.partial(
        pl.kernel,
        mesh=mesh,
        out_type=jax.ShapeDtypeStruct((S, D, B), jnp.float32),
        scratch_types=[
            pltpu.VMEM((FW,), jnp.int32),
            pltpu.VMEM((BW,), jnp.int32),
            pltpu.VMEM((BW, D), jnp.float32),
            pltpu.VMEM((D, BW), jnp.float32),
            pltpu.SemaphoreType.DMA,
        ],
        compiler_params=pltpu.CompilerParams(
            use_tc_tiling_on_sc=False, needs_layout_passes=False
        ),
    )
    def gather_kernel(idx_hbm, table_hbm, out_hbm, ids_v, sidx_v, d_v, e_v, sem):
        wid = lax.axis_index("s") * NC + lax.axis_index("c")
        b0 = wid * BW
        pltpu.sync_copy(idx_hbm.at[pl.ds(wid * FW, FW)], ids_v)

        iota = lax.iota(jnp.int32, 16)

        def body(s, carry):
            def cbody(g, c):
                rvec = iota + 16 * g
                vis = plsc.load_gather(ids_v, [rvec * S + s])
                plsc.store_scatter(sidx_v, [rvec], vis)
                return c

            lax.fori_loop(0, BW // 16, cbody, 0)
            pltpu.async_copy(table_hbm.at[sidx_v], d_v, sem).wait()

            def tbody(g, c):
                rvec = iota + 16 * g
                for d0 in range(D):
                    lvec = jnp.bitwise_and(iota + d0, D - 1)
                    x = plsc.load_gather(d_v, [rvec, lvec])
                    plsc.store_scatter(e_v, [lvec, rvec], x)
                return c

            lax.fori_loop(0, BW // 16, tbody, 0)
            pltpu.sync_copy(e_v, out_hbm.at[s].at[:, pl.ds(b0, BW)])
            return carry

        lax.fori_loop(0, S, body, 0)

    return gather_kernel


_GATHER = _make_kernel()


def kernel(element_ids, weight):
    idx = element_ids.reshape(-1)
    out = _GATHER(idx, weight)
    return out.transpose(2, 0, 1)


# speedup vs baseline: 2.1561x; 1.0826x over previous
import functools

import jax
import jax.numpy as jnp
from jax import lax
from jax.experimental import pallas as pl
from jax.experimental.pallas import tpu as pltpu
from jax.experimental.pallas import tpu_sc as plsc

D = 32
B = 16384
S = 50
NC, NS = 2, 16
NW = NC * NS
BW = B // NW
FW = BW * S
G = BW // 16


def _make_kernel():
    mesh = plsc.VectorSubcoreMesh(core_axis_name="c", subcore_axis_name="s")

    @functools.partial(
        pl.kernel,
        mesh=mesh,
        out_type=jax.ShapeDtypeStruct((S, D, B), jnp.float32),
        scratch_types=[
            pltpu.VMEM((FW,), jnp.int32),
            pltpu.VMEM((BW,), jnp.int32),
            pltpu.VMEM((BW,), jnp.int32),
            pltpu.VMEM((BW, D), jnp.float32),
            pltpu.VMEM((BW, D), jnp.float32),
            pltpu.VMEM((D, BW), jnp.float32),
            pltpu.SemaphoreType.DMA,
            pltpu.SemaphoreType.DMA,
        ],
        compiler_params=pltpu.CompilerParams(
            use_tc_tiling_on_sc=False, needs_layout_passes=False
        ),
    )
    def gather_kernel(idx_hbm, table_hbm, out_hbm, ids_v,
                      sidx0, sidx1, d0, d1, e_v, sem0, sem1):
        wid = lax.axis_index("s") * NC + lax.axis_index("c")
        b0 = wid * BW
        pltpu.sync_copy(idx_hbm.at[pl.ds(wid * FW, FW)], ids_v)

        iota = lax.iota(jnp.int32, 16)

        def compact(s, sidx):
            def cbody(g, c):
                rvec = iota + 16 * g
                vis = plsc.load_gather(ids_v, [rvec * S + s])
                plsc.store_scatter(sidx, [rvec], vis)
                return c

            lax.fori_loop(0, G, cbody, 0)

        def transpose_out(s, d_v):
            def tbody(g, c):
                rvec = iota + 16 * g
                for dd in range(D):
                    lvec = jnp.bitwise_and(iota + dd, D - 1)
                    x = plsc.load_gather(d_v, [rvec, lvec])
                    plsc.store_scatter(e_v, [lvec, rvec], x)
                return c

            lax.fori_loop(0, G, tbody, 0)
            pltpu.sync_copy(e_v, out_hbm.at[s].at[:, pl.ds(b0, BW)])

        compact(0, sidx0)
        pltpu.async_copy(table_hbm.at[sidx0], d0, sem0)

        def body(tt, carry):
            s = 2 * tt
            compact(s + 1, sidx1)
            pltpu.async_copy(table_hbm.at[sidx1], d1, sem1)
            pltpu.make_async_copy(table_hbm.at[sidx0], d0, sem0).wait()
            transpose_out(s, d0)
            compact(s + 2, sidx0)
            pltpu.async_copy(table_hbm.at[sidx0], d0, sem0)
            pltpu.make_async_copy(table_hbm.at[sidx1], d1, sem1).wait()
            transpose_out(s + 1, d1)
            return carry

        lax.fori_loop(0, S // 2 - 1, body, 0)
        compact(S - 1, sidx1)
        pltpu.async_copy(table_hbm.at[sidx1], d1, sem1)
        pltpu.make_async_copy(table_hbm.at[sidx0], d0, sem0).wait()
        transpose_out(S - 2, d0)
        pltpu.make_async_copy(table_hbm.at[sidx1], d1, sem1).wait()
        transpose_out(S - 1, d1)

    return gather_kernel


_GATHER = _make_kernel()


def kernel(element_ids, weight):
    idx = element_ids.reshape(-1)
    out = _GATHER(idx, weight)
    return out.transpose(2, 0, 1)


# speedup vs baseline: 2.2387x; 1.0383x over previous
import functools

import jax
import jax.numpy as jnp
from jax import lax
from jax.experimental import pallas as pl
from jax.experimental.pallas import tpu as pltpu
from jax.experimental.pallas import tpu_sc as plsc

D = 32
B = 16384
S = 50
NC, NS = 2, 16
NW = NC * NS
BW = B // NW
FW = BW * S
G = BW // 16


def _make_kernel():
    mesh = plsc.VectorSubcoreMesh(core_axis_name="c", subcore_axis_name="s")

    @functools.partial(
        pl.kernel,
        mesh=mesh,
        out_type=jax.ShapeDtypeStruct((S, D, B), jnp.float32),
        scratch_types=[
            pltpu.VMEM((FW,), jnp.int32),
            pltpu.VMEM((BW,), jnp.int32),
            pltpu.VMEM((BW,), jnp.int32),
            pltpu.VMEM((BW, D), jnp.float32),
            pltpu.VMEM((BW, D), jnp.float32),
            pltpu.VMEM((D, BW), jnp.float32),
            pltpu.VMEM((D, BW), jnp.float32),
            pltpu.SemaphoreType.DMA,
            pltpu.SemaphoreType.DMA,
            pltpu.SemaphoreType.DMA,
            pltpu.SemaphoreType.DMA,
        ],
        compiler_params=pltpu.CompilerParams(
            use_tc_tiling_on_sc=False, needs_layout_passes=False
        ),
    )
    def gather_kernel(idx_hbm, table_hbm, out_hbm, ids_v,
                      sidx0, sidx1, d0, d1, e0, e1, sem0, sem1, wsem0, wsem1):
        wid = lax.axis_index("s") * NC + lax.axis_index("c")
        b0 = wid * BW
        pltpu.sync_copy(idx_hbm.at[pl.ds(wid * FW, FW)], ids_v)

        iota = lax.iota(jnp.int32, 16)

        def out_slice(s):
            return out_hbm.at[s].at[:, pl.ds(b0, BW)]

        def compact(s, sidx):
            def cbody(g, c):
                rvec = iota + 16 * g
                vis = plsc.load_gather(ids_v, [rvec * S + s])
                plsc.store_scatter(sidx, [rvec], vis)
                return c

            lax.fori_loop(0, G, cbody, 0)

        def transpose(d_v, e_v):
            def tbody(g, c):
                rvec = iota + 16 * g
                for dd in range(D):
                    lvec = jnp.bitwise_and(iota + dd, D - 1)
                    x = plsc.load_gather(d_v, [rvec, lvec])
                    plsc.store_scatter(e_v, [lvec, rvec], x)
                return c

            lax.fori_loop(0, G, tbody, 0)

        compact(0, sidx0)
        pltpu.async_copy(table_hbm.at[sidx0], d0, sem0)

        def body(tt, carry):
            s = 2 * tt
            compact(s + 1, sidx1)
            pltpu.async_copy(table_hbm.at[sidx1], d1, sem1)
            pltpu.make_async_copy(table_hbm.at[sidx0], d0, sem0).wait()

            @pl.when(tt > 0)
            def _():
                pltpu.make_async_copy(e0, out_slice(0), wsem0).wait()

            transpose(d0, e0)
            pltpu.async_copy(e0, out_slice(s), wsem0)
            compact(s + 2, sidx0)
            pltpu.async_copy(table_hbm.at[sidx0], d0, sem0)
            pltpu.make_async_copy(table_hbm.at[sidx1], d1, sem1).wait()

            @pl.when(tt > 0)
            def _():
                pltpu.make_async_copy(e1, out_slice(0), wsem1).wait()

            transpose(d1, e1)
            pltpu.async_copy(e1, out_slice(s + 1), wsem1)
            return carry

        lax.fori_loop(0, S // 2 - 1, body, 0)
        compact(S - 1, sidx1)
        pltpu.async_copy(table_hbm.at[sidx1], d1, sem1)
        pltpu.make_async_copy(table_hbm.at[sidx0], d0, sem0).wait()
        pltpu.make_async_copy(e0, out_slice(0), wsem0).wait()
        transpose(d0, e0)
        pltpu.async_copy(e0, out_slice(S - 2), wsem0)
        pltpu.make_async_copy(table_hbm.at[sidx1], d1, sem1).wait()
        pltpu.make_async_copy(e1, out_slice(0), wsem1).wait()
        transpose(d1, e1)
        pltpu.async_copy(e1, out_slice(S - 1), wsem1)
        pltpu.make_async_copy(e0, out_slice(0), wsem0).wait()
        pltpu.make_async_copy(e1, out_slice(0), wsem1).wait()

    return gather_kernel


_GATHER = _make_kernel()


def kernel(element_ids, weight):
    idx = element_ids.reshape(-1)
    out = _GATHER(idx, weight)
    return out.transpose(2, 0, 1)


# speedup vs baseline: 2.2655x; 1.0120x over previous
import functools

import jax
import jax.numpy as jnp
from jax import lax
from jax.experimental import pallas as pl
from jax.experimental.pallas import tpu as pltpu
from jax.experimental.pallas import tpu_sc as plsc

D = 32
B = 16384
S = 50
NC, NS = 2, 16
NW = NC * NS
BW = B // NW
FW = BW * S
G = BW // 16


def _make_kernel():
    mesh = plsc.VectorSubcoreMesh(core_axis_name="c", subcore_axis_name="s")

    @functools.partial(
        pl.kernel,
        mesh=mesh,
        out_type=jax.ShapeDtypeStruct((S, D, B), jnp.float32),
        scratch_types=[
            pltpu.VMEM((FW,), jnp.int32),
            pltpu.VMEM((BW,), jnp.int32),
            pltpu.VMEM((BW,), jnp.int32),
            pltpu.VMEM((BW, D), jnp.float32),
            pltpu.VMEM((BW, D), jnp.float32),
            pltpu.VMEM((D, BW + 1), jnp.float32),
            pltpu.VMEM((D, BW + 1), jnp.float32),
            pltpu.SemaphoreType.DMA,
            pltpu.SemaphoreType.DMA,
            pltpu.SemaphoreType.DMA,
            pltpu.SemaphoreType.DMA,
        ],
        compiler_params=pltpu.CompilerParams(
            use_tc_tiling_on_sc=False, needs_layout_passes=False
        ),
    )
    def gather_kernel(idx_hbm, table_hbm, out_hbm, ids_v,
                      sidx0, sidx1, d0, d1, e0, e1, sem0, sem1, wsem0, wsem1):
        wid = lax.axis_index("s") * NC + lax.axis_index("c")
        b0 = wid * BW
        pltpu.sync_copy(idx_hbm.at[pl.ds(wid * FW, FW)], ids_v)

        iota = lax.iota(jnp.int32, 16)

        def out_slice(s):
            return out_hbm.at[s].at[:, pl.ds(b0, BW)]

        def compact(s, sidx):
            def cbody(g, c):
                rvec = iota + 16 * g
                vis = plsc.load_gather(ids_v, [rvec * S + s])
                plsc.store_scatter(sidx, [rvec], vis)
                return c

            lax.fori_loop(0, G, cbody, 0)

        iota_hi = iota + 16

        def transpose(d_v, e_v):
            def tbody(g, c):
                for j in range(16):
                    r = 16 * g + j
                    rs = lax.broadcast(r, (16,))
                    x0 = d_v[r, pl.ds(0, 16)]
                    plsc.store_scatter(e_v, [iota, rs], x0)
                    x1 = d_v[r, pl.ds(16, 16)]
                    plsc.store_scatter(e_v, [iota_hi, rs], x1)
                return c

            lax.fori_loop(0, G, tbody, 0)

        compact(0, sidx0)
        pltpu.async_copy(table_hbm.at[sidx0], d0, sem0)

        def body(tt, carry):
            s = 2 * tt
            compact(s + 1, sidx1)
            pltpu.async_copy(table_hbm.at[sidx1], d1, sem1)
            pltpu.make_async_copy(table_hbm.at[sidx0], d0, sem0).wait()

            @pl.when(tt > 0)
            def _():
                pltpu.make_async_copy(e0.at[:, pl.ds(0, BW)], out_slice(0), wsem0).wait()

            transpose(d0, e0)
            pltpu.async_copy(e0.at[:, pl.ds(0, BW)], out_slice(s), wsem0)
            compact(s + 2, sidx0)
            pltpu.async_copy(table_hbm.at[sidx0], d0, sem0)
            pltpu.make_async_copy(table_hbm.at[sidx1], d1, sem1).wait()

            @pl.when(tt > 0)
            def _():
                pltpu.make_async_copy(e1.at[:, pl.ds(0, BW)], out_slice(0), wsem1).wait()

            transpose(d1, e1)
            pltpu.async_copy(e1.at[:, pl.ds(0, BW)], out_slice(s + 1), wsem1)
            return carry

        lax.fori_loop(0, S // 2 - 1, body, 0)
        compact(S - 1, sidx1)
        pltpu.async_copy(table_hbm.at[sidx1], d1, sem1)
        pltpu.make_async_copy(table_hbm.at[sidx0], d0, sem0).wait()
        pltpu.make_async_copy(e0.at[:, pl.ds(0, BW)], out_slice(0), wsem0).wait()
        transpose(d0, e0)
        pltpu.async_copy(e0.at[:, pl.ds(0, BW)], out_slice(S - 2), wsem0)
        pltpu.make_async_copy(table_hbm.at[sidx1], d1, sem1).wait()
        pltpu.make_async_copy(e1.at[:, pl.ds(0, BW)], out_slice(0), wsem1).wait()
        transpose(d1, e1)
        pltpu.async_copy(e1.at[:, pl.ds(0, BW)], out_slice(S - 1), wsem1)
        pltpu.make_async_copy(e0.at[:, pl.ds(0, BW)], out_slice(0), wsem0).wait()
        pltpu.make_async_copy(e1.at[:, pl.ds(0, BW)], out_slice(0), wsem1).wait()

    return gather_kernel


_GATHER = _make_kernel()


def kernel(element_ids, weight):
    idx = element_ids.reshape(-1)
    out = _GATHER(idx, weight)
    return out.transpose(2, 0, 1)


# speedup vs baseline: 2.6354x; 1.1633x over previous
import functools

import jax
import jax.numpy as jnp
from jax import lax
from jax.experimental import pallas as pl
from jax.experimental.pallas import tpu as pltpu
from jax.experimental.pallas import tpu_sc as plsc

D = 32
B = 16384
S = 50
NC, NS = 2, 16
NW = NC * NS
BW = B // NW
FW = BW * S
G = BW // 16


def _make_kernel():
    mesh = plsc.VectorSubcoreMesh(core_axis_name="c", subcore_axis_name="s")

    @functools.partial(
        pl.kernel,
        mesh=mesh,
        out_type=jax.ShapeDtypeStruct((S, D, B), jnp.float32),
        scratch_types=[
            pltpu.VMEM((FW,), jnp.int32),
            pltpu.VMEM((BW,), jnp.int32),
            pltpu.VMEM((BW,), jnp.int32),
            pltpu.VMEM((BW, D), jnp.float32),
            pltpu.VMEM((BW, D), jnp.float32),
            pltpu.VMEM((D, BW + 1), jnp.float32),
            pltpu.VMEM((D, BW + 1), jnp.float32),
            pltpu.SemaphoreType.DMA,
            pltpu.SemaphoreType.DMA,
            pltpu.SemaphoreType.DMA,
            pltpu.SemaphoreType.DMA,
        ],
        compiler_params=pltpu.CompilerParams(
            use_tc_tiling_on_sc=False, needs_layout_passes=False
        ),
    )
    def gather_kernel(idx_hbm, table_hbm, out_hbm, ids_v,
                      sidx0, sidx1, d0, d1, e0, e1, sem0, sem1, wsem0, wsem1):
        wid = lax.axis_index("s") * NC + lax.axis_index("c")
        b0 = wid * BW
        pltpu.sync_copy(idx_hbm.at[pl.ds(wid * FW, FW)], ids_v)

        iota = lax.iota(jnp.int32, 16)

        def out_slice(s):
            return out_hbm.at[s].at[:, pl.ds(b0, BW)]

        def compact(s, sidx):
            def cbody(g, c):
                rvec = iota + 16 * g
                vis = plsc.load_gather(ids_v, [rvec * S + s])
                plsc.store_scatter(sidx, [rvec], vis)
                return c

            lax.fori_loop(0, G, cbody, 0)

        iota_hi = iota + 16

        def transpose(d_v, e_v):
            @plsc.parallel_loop(0, BW, step=8, unroll=2)
            def tbody(r0):
                for j in range(8):
                    r = r0 + j
                    rs = lax.broadcast(r, (16,))
                    x0 = d_v[r, pl.ds(0, 16)]
                    plsc.store_scatter(e_v, [iota, rs], x0)
                    x1 = d_v[r, pl.ds(16, 16)]
                    plsc.store_scatter(e_v, [iota_hi, rs], x1)

        compact(0, sidx0)
        pltpu.async_copy(table_hbm.at[sidx0], d0, sem0)

        def body(tt, carry):
            s = 2 * tt
            compact(s + 1, sidx1)
            pltpu.async_copy(table_hbm.at[sidx1], d1, sem1)
            pltpu.make_async_copy(table_hbm.at[sidx0], d0, sem0).wait()

            @pl.when(tt > 0)
            def _():
                pltpu.make_async_copy(e0.at[:, pl.ds(0, BW)], out_slice(0), wsem0).wait()

            transpose(d0, e0)
            pltpu.async_copy(e0.at[:, pl.ds(0, BW)], out_slice(s), wsem0)
            compact(s + 2, sidx0)
            pltpu.async_copy(table_hbm.at[sidx0], d0, sem0)
            pltpu.make_async_copy(table_hbm.at[sidx1], d1, sem1).wait()

            @pl.when(tt > 0)
            def _():
                pltpu.make_async_copy(e1.at[:, pl.ds(0, BW)], out_slice(0), wsem1).wait()

            transpose(d1, e1)
            pltpu.async_copy(e1.at[:, pl.ds(0, BW)], out_slice(s + 1), wsem1)
            return carry

        lax.fori_loop(0, S // 2 - 1, body, 0)
        compact(S - 1, sidx1)
        pltpu.async_copy(table_hbm.at[sidx1], d1, sem1)
        pltpu.make_async_copy(table_hbm.at[sidx0], d0, sem0).wait()
        pltpu.make_async_copy(e0.at[:, pl.ds(0, BW)], out_slice(0), wsem0).wait()
        transpose(d0, e0)
        pltpu.async_copy(e0.at[:, pl.ds(0, BW)], out_slice(S - 2), wsem0)
        pltpu.make_async_copy(table_hbm.at[sidx1], d1, sem1).wait()
        pltpu.make_async_copy(e1.at[:, pl.ds(0, BW)], out_slice(0), wsem1).wait()
        transpose(d1, e1)
        pltpu.async_copy(e1.at[:, pl.ds(0, BW)], out_slice(S - 1), wsem1)
        pltpu.make_async_copy(e0.at[:, pl.ds(0, BW)], out_slice(0), wsem0).wait()
        pltpu.make_async_copy(e1.at[:, pl.ds(0, BW)], out_slice(0), wsem1).wait()

    return gather_kernel


_GATHER = _make_kernel()


def kernel(element_ids, weight):
    idx = element_ids.reshape(-1)
    out = _GATHER(idx, weight)
    return out.transpose(2, 0, 1)
